# baseline (speedup 1.0000x reference)
import functools

import jax
import jax.numpy as jnp
from jax import lax
from jax.experimental import pallas as pl
from jax.experimental.pallas import tpu as pltpu
from jax.experimental.pallas import tpu_sc as plsc

_PERCENT_T = 0.02
_ROWS_PER_BLOCK = 8
_H_BLK = 96

_NBINS = 32768
_BIN_SHIFT = 17
_CHUNK_H = 48


def _sc_stats_kernel(x_hbm, out_hbm, buf, counts, sums, outbuf, *, k, rows_per_worker, h, w):
    cid = lax.axis_index("c")
    sid = lax.axis_index("s")
    wid = sid * 2 + cid
    n_chunks = h // _CHUNK_H
    vecs_per_row = (_CHUNK_H * w) // 16
    ki = jnp.int32(k)

    def row_body(rr, _):
        row = wid * rows_per_worker + rr

        def zero_body(i, _):
            counts[pl.ds(i * 16, 16)] = jnp.zeros((16,), jnp.int32)
            sums[pl.ds(i * 16, 16)] = jnp.zeros((16,), jnp.float32)
            return 0

        lax.fori_loop(0, _NBINS // 16, zero_body, 0)

        def chunk_body(c, carry):
            runmax, binmax = carry
            pltpu.sync_copy(x_hbm.at[row, pl.ds(c * _CHUNK_H, _CHUNK_H), :], buf)

            def vec_body(t, carry2):
                rm, bm = carry2
                i = t // (w // 16)
                j = t % (w // 16)
                v = buf[i, pl.ds(j * 16, 16)]
                u = plsc.bitcast(v, jnp.int32)
                s = u ^ ((u >> 31) | jnp.int32(-2147483648))
                bin_ = lax.shift_right_logical(s, _BIN_SHIFT)
                plsc.addupdate_scatter(counts, [bin_], jnp.ones((16,), jnp.int32))
                plsc.addupdate_scatter(sums, [bin_], v)
                return jnp.maximum(rm, v), jnp.maximum(bm, bin_)

            return lax.fori_loop(0, vecs_per_row, vec_body, (runmax, binmax))

        init = (jnp.full((16,), -3.0e38, jnp.float32), jnp.zeros((16,), jnp.int32))
        runmax, binmax = lax.fori_loop(0, n_chunks, chunk_body, init)
        rowmax = lax.reduce_max(runmax, (0,))
        g_start = lax.reduce_max(binmax, (0,)) // 16

        def scan_cond(st):
            g, cnt, sm, pc, ps = st
            return cnt < ki

        def scan_body(st):
            g, cnt, sm, pc, ps = st
            cv = counts[pl.ds(g * 16, 16)]
            sv = sums[pl.ds(g * 16, 16)]
            return (g - 1,
                    cnt + lax.reduce_sum(cv, (0,)),
                    sm + lax.reduce_sum(sv, (0,)),
                    cnt, sm)

        st0 = (g_start, jnp.int32(0), jnp.float32(0.0), jnp.int32(0), jnp.float32(0.0))
        g_end, cnt_t, sm_t, prev_c, prev_s = lax.while_loop(scan_cond, scan_body, st0)

        gb = g_end + 1
        cv = counts[pl.ds(gb * 16, 16)]
        sv = sums[pl.ds(gb * 16, 16)]
        rcv = lax.rev(cv, (0,))
        rsv = lax.rev(sv, (0,))
        ccum = lax.cumsum(rcv, axis=0)
        scum = lax.cumsum(rsv, axis=0)
        crossed = (prev_c + ccum) >= ki
        t = jnp.int32(16) - plsc.all_reduce_population_count(crossed)
        onehot = lax.iota(jnp.int32, 16) == t

        def sel_i(vec):
            return lax.reduce_sum(jnp.where(onehot, vec, 0), (0,))

        def sel_f(vec):
            return lax.reduce_sum(jnp.where(onehot, vec, 0.0), (0,))

        cb = sel_i(rcv)
        sb = sel_f(rsv)
        m_above = prev_c + sel_i(ccum) - cb
        s_above = prev_s + sel_f(scum) - sb
        k2 = (ki - m_above).astype(jnp.float32)

        bmv = jnp.full((16,), sb) / jnp.full((16,), cb.astype(jnp.float32))
        avg_v = (jnp.full((16,), s_above) + jnp.full((16,), k2) * bmv) * (
            jnp.float32(1.0 / k))
        lanes = lax.iota(jnp.int32, 16)
        dbg = [avg_v,
               jnp.full((16,), rowmax),
               jnp.full((16,), g_start.astype(jnp.float32)),
               jnp.full((16,), (g_end + 1).astype(jnp.float32)),
               jnp.full((16,), prev_c.astype(jnp.float32)),
               jnp.full((16,), prev_s),
               jnp.full((16,), cnt_t.astype(jnp.float32)),
               jnp.full((16,), sm_t),
               jnp.full((16,), cb.astype(jnp.float32)),
               jnp.full((16,), sb),
               jnp.full((16,), m_above.astype(jnp.float32)),
               jnp.full((16,), s_above),
               jnp.full((16,), k2)]
        out_v = jnp.zeros((16,), jnp.float32)
        for ln, val in enumerate(dbg):
            out_v = jnp.where(lanes == ln, val, out_v)
        outbuf[...] = out_v
        pltpu.sync_copy(outbuf, out_hbm.at[row])
        return 0

    lax.fori_loop(0, rows_per_worker, row_body, 0)


def _mlp_kernel(avg_ref, max_ref, w1_ref, b1_ref, w2_ref, b2_ref, scale_ref):
    w1 = w1_ref[...]
    b1 = b1_ref[...]
    w2 = w2_ref[...]
    b2 = b2_ref[...]

    def mlp(p):
        h = jnp.dot(p, w1.T, preferred_element_type=jnp.float32) + b1
        h = jnp.maximum(h, 0.0)
        return jnp.dot(h, w2.T, preferred_element_type=jnp.float32) + b2

    att = mlp(avg_ref[...]) + mlp(max_ref[...])
    scale_ref[...] = jax.nn.sigmoid(att)


def _scale_kernel(x_ref, s_ref, o_ref):
    o_ref[...] = x_ref[...] * s_ref[...]


def kernel(x, W1, b1, W2, b2):
    B, C, H, Wd = x.shape
    N = H * Wd
    R = B * C
    k = int(round(N * _PERCENT_T))
    x3 = x.reshape(R, H, Wd)

    n_workers = 32
    rows_per_worker = R // n_workers
    mesh = plsc.VectorSubcoreMesh(core_axis_name="c", subcore_axis_name="s")
    sc_stats = functools.partial(
        pl.kernel,
        mesh=mesh,
        compiler_params=pltpu.CompilerParams(needs_layout_passes=False),
        out_type=jax.ShapeDtypeStruct((R, 16), jnp.float32),
        scratch_types=[
            pltpu.VMEM((_CHUNK_H, Wd), jnp.float32),
            pltpu.VMEM((_NBINS,), jnp.int32),
            pltpu.VMEM((_NBINS,), jnp.float32),
            pltpu.VMEM((16,), jnp.float32),
        ],
    )(functools.partial(_sc_stats_kernel, k=k, rows_per_worker=rows_per_worker,
                        h=H, w=Wd))
    pools = sc_stats(x3)

    avg_pool = pools[:, 0].reshape(B, C)
    max_pool = pools[:, 1].reshape(B, C)

    scale = pl.pallas_call(
        _mlp_kernel,
        out_shape=jax.ShapeDtypeStruct((B, C), jnp.float32),
    )(avg_pool, max_pool, W1, b1.reshape(1, -1), W2, b2.reshape(1, -1))

    scale3 = scale.reshape(R, 1, 1)
    hb = _H_BLK if H % _H_BLK == 0 else H
    rows = _ROWS_PER_BLOCK
    y = pl.pallas_call(
        _scale_kernel,
        grid=(R // rows, H // hb),
        in_specs=[
            pl.BlockSpec((rows, hb, Wd), lambda i, j: (i, j, 0)),
            pl.BlockSpec((rows, 1, 1), lambda i, j: (i, 0, 0)),
        ],
        out_specs=pl.BlockSpec((rows, hb, Wd), lambda i, j: (i, j, 0)),
        out_shape=jax.ShapeDtypeStruct((R, H, Wd), jnp.float32),
    )(x3, scale3)

    return y.reshape(B, C, H, Wd)


# speedup vs baseline: 1.1526x; 1.1526x over previous
import functools

import jax
import jax.numpy as jnp
from jax import lax
from jax.experimental import pallas as pl
from jax.experimental.pallas import tpu as pltpu
from jax.experimental.pallas import tpu_sc as plsc

_PERCENT_T = 0.02
_ROWS_PER_BLOCK = 8
_H_BLK = 96

_NBINS = 8192
_BIN_SHIFT = 19
_CHUNK_H = 96


def _sc_stats_kernel(x_hbm, out_hbm, buf, counts, sums, outbuf, sem0, sem1,
                     *, k, rows_per_worker, h, w):
    cid = lax.axis_index("c")
    sid = lax.axis_index("s")
    wid = sid * 2 + cid
    n_chunks = h // _CHUNK_H
    ki = jnp.int32(k)
    ones_i = jnp.ones((16,), jnp.int32)

    def row_body(rr, _):
        row = wid * rows_per_worker + rr

        def dma(c, bref, sem):
            return pltpu.make_async_copy(
                x_hbm.at[row, pl.ds(c * _CHUNK_H, _CHUNK_H), :], bref, sem)

        def zero_body(i):
            counts[pl.ds(i * 16, 16)] = jnp.zeros((16,), jnp.int32)
            sums[pl.ds(i * 16, 16)] = jnp.zeros((16,), jnp.float32)

        plsc.parallel_loop(0, _NBINS // 16, unroll=4)(zero_body)

        def proc(bref, carry):
            def outer_i(i, car):
                def inner(j, c2):
                    rm, bm = c2
                    v = bref[i, pl.ds(j * 16, 16)]
                    u = plsc.bitcast(v, jnp.int32)
                    s = u ^ ((u >> 31) | jnp.int32(-2147483648))
                    bin_ = lax.shift_right_logical(s, _BIN_SHIFT)
                    plsc.addupdate_scatter(counts, [bin_], ones_i)
                    plsc.addupdate_scatter(sums, [bin_], v)
                    return (jnp.maximum(rm, v), jnp.maximum(bm, bin_))

                return lax.fori_loop(0, w // 16, inner, car)

            return lax.fori_loop(0, _CHUNK_H, outer_i, carry)

        dma(0, buf.at[0], sem0).start()

        def pair_body(p, carry):
            base = 2 * p
            dma(base + 1, buf.at[1], sem1).start()
            dma(base, buf.at[0], sem0).wait()
            carry = proc(buf.at[0], carry)

            @pl.when(base + 2 < n_chunks)
            def _():
                dma(base + 2, buf.at[0], sem0).start()

            dma(base + 1, buf.at[1], sem1).wait()
            return proc(buf.at[1], carry)

        init = (jnp.full((16,), -3.0e38, jnp.float32),
                jnp.zeros((16,), jnp.int32))
        runmax, binmax = lax.fori_loop(0, n_chunks // 2, pair_body, init)
        rowmax = lax.reduce_max(runmax, (0,))
        g_start = lax.reduce_max(binmax, (0,)) // 16

        def scan_cond(st):
            g, cnt, sm, pc, ps = st
            return cnt < ki

        def scan_body(st):
            g, cnt, sm, pc, ps = st
            cv = counts[pl.ds(g * 16, 16)]
            sv = sums[pl.ds(g * 16, 16)]
            return (g - 1,
                    cnt + lax.reduce_sum(cv, (0,)),
                    sm + lax.reduce_sum(sv, (0,)),
                    cnt, sm)

        st0 = (g_start, jnp.int32(0), jnp.float32(0.0), jnp.int32(0), jnp.float32(0.0))
        g_end, cnt_t, sm_t, prev_c, prev_s = lax.while_loop(scan_cond, scan_body, st0)

        gb = g_end + 1
        cv = counts[pl.ds(gb * 16, 16)]
        sv = sums[pl.ds(gb * 16, 16)]
        rcv = lax.rev(cv, (0,))
        rsv = lax.rev(sv, (0,))
        ccum = lax.cumsum(rcv, axis=0)
        scum = lax.cumsum(rsv, axis=0)
        crossed = (prev_c + ccum) >= ki
        t = jnp.int32(16) - plsc.all_reduce_population_count(crossed)
        onehot = lax.iota(jnp.int32, 16) == t

        def sel_i(vec):
            return lax.reduce_sum(jnp.where(onehot, vec, 0), (0,))

        def sel_f(vec):
            return lax.reduce_sum(jnp.where(onehot, vec, 0.0), (0,))

        cb = sel_i(rcv)
        sb = sel_f(rsv)
        m_above = prev_c + sel_i(ccum) - cb
        s_above = prev_s + sel_f(scum) - sb
        k2 = (ki - m_above).astype(jnp.float32)

        bmv = jnp.full((16,), sb) / jnp.full((16,), cb.astype(jnp.float32))
        avg_v = (jnp.full((16,), s_above) + jnp.full((16,), k2) * bmv) * (
            jnp.float32(1.0 / k))
        lanes = lax.iota(jnp.int32, 16)
        dbg = [avg_v,
               jnp.full((16,), rowmax),
               jnp.full((16,), g_start.astype(jnp.float32)),
               jnp.full((16,), (g_end + 1).astype(jnp.float32)),
               jnp.full((16,), prev_c.astype(jnp.float32)),
               jnp.full((16,), prev_s),
               jnp.full((16,), cnt_t.astype(jnp.float32)),
               jnp.full((16,), sm_t),
               jnp.full((16,), cb.astype(jnp.float32)),
               jnp.full((16,), sb),
               jnp.full((16,), m_above.astype(jnp.float32)),
               jnp.full((16,), s_above),
               jnp.full((16,), k2)]
        out_v = jnp.zeros((16,), jnp.float32)
        for ln, val in enumerate(dbg):
            out_v = jnp.where(lanes == ln, val, out_v)
        outbuf[...] = out_v
        pltpu.sync_copy(outbuf, out_hbm.at[row])
        return 0

    lax.fori_loop(0, rows_per_worker, row_body, 0)


def _make_sc_stats(R, H, Wd, k):
    mesh = plsc.VectorSubcoreMesh(core_axis_name="c", subcore_axis_name="s")
    return functools.partial(
        pl.kernel,
        mesh=mesh,
        compiler_params=pltpu.CompilerParams(needs_layout_passes=False),
        out_type=jax.ShapeDtypeStruct((R, 16), jnp.float32),
        scratch_types=[
            pltpu.VMEM((2, _CHUNK_H, Wd), jnp.float32),
            pltpu.VMEM((_NBINS,), jnp.int32),
            pltpu.VMEM((_NBINS,), jnp.float32),
            pltpu.VMEM((16,), jnp.float32),
            pltpu.SemaphoreType.DMA,
            pltpu.SemaphoreType.DMA,
        ],
    )(functools.partial(_sc_stats_kernel, k=k, rows_per_worker=R // 32,
                        h=H, w=Wd))


def _mlp_kernel(avg_ref, max_ref, w1_ref, b1_ref, w2_ref, b2_ref, scale_ref):
    w1 = w1_ref[...]
    b1 = b1_ref[...]
    w2 = w2_ref[...]
    b2 = b2_ref[...]

    def mlp(p):
        h = jnp.dot(p, w1.T, preferred_element_type=jnp.float32) + b1
        h = jnp.maximum(h, 0.0)
        return jnp.dot(h, w2.T, preferred_element_type=jnp.float32) + b2

    att = mlp(avg_ref[...]) + mlp(max_ref[...])
    scale_ref[...] = jax.nn.sigmoid(att)


def _scale_kernel(x_ref, s_ref, o_ref):
    o_ref[...] = x_ref[...] * s_ref[...]


def kernel(x, W1, b1, W2, b2):
    B, C, H, Wd = x.shape
    N = H * Wd
    R = B * C
    k = int(round(N * _PERCENT_T))
    x3 = x.reshape(R, H, Wd)

    pools = _make_sc_stats(R, H, Wd, k)(x3)

    avg_pool = pools[:, 0].reshape(B, C)
    max_pool = pools[:, 1].reshape(B, C)

    scale = pl.pallas_call(
        _mlp_kernel,
        out_shape=jax.ShapeDtypeStruct((B, C), jnp.float32),
    )(avg_pool, max_pool, W1, b1.reshape(1, -1), W2, b2.reshape(1, -1))

    scale3 = scale.reshape(R, 1, 1)
    hb = _H_BLK if H % _H_BLK == 0 else H
    rows = _ROWS_PER_BLOCK
    y = pl.pallas_call(
        _scale_kernel,
        grid=(R // rows, H // hb),
        in_specs=[
            pl.BlockSpec((rows, hb, Wd), lambda i, j: (i, j, 0)),
            pl.BlockSpec((rows, 1, 1), lambda i, j: (i, 0, 0)),
        ],
        out_specs=pl.BlockSpec((rows, hb, Wd), lambda i, j: (i, j, 0)),
        out_shape=jax.ShapeDtypeStruct((R, H, Wd), jnp.float32),
    )(x3, scale3)

    return y.reshape(B, C, H, Wd)


# speedup vs baseline: 1.1669x; 1.0124x over previous
import functools

import jax
import jax.numpy as jnp
from jax import lax
from jax.experimental import pallas as pl
from jax.experimental.pallas import tpu as pltpu
from jax.experimental.pallas import tpu_sc as plsc

_PERCENT_T = 0.02
_ROWS_PER_BLOCK = 8
_H_BLK = 96

_NBINS = 8192
_BIN_SHIFT = 19
_CHUNK_H = 96


def _sc_stats_kernel(x_hbm, out_hbm, buf, counts, sums, outbuf, sem0, sem1,
                     *, k, rows_per_worker, h, w):
    cid = lax.axis_index("c")
    sid = lax.axis_index("s")
    wid = sid * 2 + cid
    n_chunks = h // _CHUNK_H
    ki = jnp.int32(k)
    ones_i = jnp.ones((16,), jnp.int32)

    def row_body(rr, _):
        row = wid * rows_per_worker + rr

        def dma(c, bref, sem):
            return pltpu.make_async_copy(
                x_hbm.at[row, pl.ds(c * _CHUNK_H, _CHUNK_H), :], bref, sem)

        def zero_body(i):
            counts[pl.ds(i * 16, 16)] = jnp.zeros((16,), jnp.int32)
            sums[pl.ds(i * 16, 16)] = jnp.zeros((16,), jnp.float32)

        plsc.parallel_loop(0, _NBINS // 16, unroll=4)(zero_body)

        def proc(bref, carry):
            def outer_i(i, car):
                rm, bm = car
                for uix in range(w // 16):
                    v = bref[i, pl.ds(uix * 16, 16)]
                    u = plsc.bitcast(v, jnp.int32)
                    s = u ^ ((u >> 31) | jnp.int32(-2147483648))
                    bin_ = lax.shift_right_logical(s, _BIN_SHIFT)
                    plsc.addupdate_scatter(counts, [bin_], ones_i)
                    plsc.addupdate_scatter(sums, [bin_], v)
                    rm = jnp.maximum(rm, v)
                    bm = jnp.maximum(bm, bin_)
                return (rm, bm)

            return lax.fori_loop(0, _CHUNK_H, outer_i, carry)

        dma(0, buf.at[0], sem0).start()

        def pair_body(p, carry):
            base = 2 * p
            dma(base + 1, buf.at[1], sem1).start()
            dma(base, buf.at[0], sem0).wait()
            carry = proc(buf.at[0], carry)

            @pl.when(base + 2 < n_chunks)
            def _():
                dma(base + 2, buf.at[0], sem0).start()

            dma(base + 1, buf.at[1], sem1).wait()
            return proc(buf.at[1], carry)

        init = (jnp.full((16,), -3.0e38, jnp.float32),
                jnp.zeros((16,), jnp.int32))
        runmax, binmax = lax.fori_loop(0, n_chunks // 2, pair_body, init)
        rowmax = lax.reduce_max(runmax, (0,))
        g_start = lax.reduce_max(binmax, (0,)) // 16

        def scan_cond(st):
            g, cnt, sm, pc, ps = st
            return cnt < ki

        def scan_body(st):
            g, cnt, sm, pc, ps = st
            cv = counts[pl.ds(g * 16, 16)]
            sv = sums[pl.ds(g * 16, 16)]
            return (g - 1,
                    cnt + lax.reduce_sum(cv, (0,)),
                    sm + lax.reduce_sum(sv, (0,)),
                    cnt, sm)

        st0 = (g_start, jnp.int32(0), jnp.float32(0.0), jnp.int32(0), jnp.float32(0.0))
        g_end, cnt_t, sm_t, prev_c, prev_s = lax.while_loop(scan_cond, scan_body, st0)

        gb = g_end + 1
        cv = counts[pl.ds(gb * 16, 16)]
        sv = sums[pl.ds(gb * 16, 16)]
        rcv = lax.rev(cv, (0,))
        rsv = lax.rev(sv, (0,))
        ccum = lax.cumsum(rcv, axis=0)
        scum = lax.cumsum(rsv, axis=0)
        crossed = (prev_c + ccum) >= ki
        t = jnp.int32(16) - plsc.all_reduce_population_count(crossed)
        onehot = lax.iota(jnp.int32, 16) == t

        def sel_i(vec):
            return lax.reduce_sum(jnp.where(onehot, vec, 0), (0,))

        def sel_f(vec):
            return lax.reduce_sum(jnp.where(onehot, vec, 0.0), (0,))

        cb = sel_i(rcv)
        sb = sel_f(rsv)
        m_above = prev_c + sel_i(ccum) - cb
        s_above = prev_s + sel_f(scum) - sb
        k2 = (ki - m_above).astype(jnp.float32)

        bmv = jnp.full((16,), sb) / jnp.full((16,), cb.astype(jnp.float32))
        avg_v = (jnp.full((16,), s_above) + jnp.full((16,), k2) * bmv) * (
            jnp.float32(1.0 / k))
        lanes = lax.iota(jnp.int32, 16)
        dbg = [avg_v,
               jnp.full((16,), rowmax),
               jnp.full((16,), g_start.astype(jnp.float32)),
               jnp.full((16,), (g_end + 1).astype(jnp.float32)),
               jnp.full((16,), prev_c.astype(jnp.float32)),
               jnp.full((16,), prev_s),
               jnp.full((16,), cnt_t.astype(jnp.float32)),
               jnp.full((16,), sm_t),
               jnp.full((16,), cb.astype(jnp.float32)),
               jnp.full((16,), sb),
               jnp.full((16,), m_above.astype(jnp.float32)),
               jnp.full((16,), s_above),
               jnp.full((16,), k2)]
        out_v = jnp.zeros((16,), jnp.float32)
        for ln, val in enumerate(dbg):
            out_v = jnp.where(lanes == ln, val, out_v)
        outbuf[...] = out_v
        pltpu.sync_copy(outbuf, out_hbm.at[row])
        return 0

    lax.fori_loop(0, rows_per_worker, row_body, 0)


def _make_sc_stats(R, H, Wd, k):
    mesh = plsc.VectorSubcoreMesh(core_axis_name="c", subcore_axis_name="s")
    return functools.partial(
        pl.kernel,
        mesh=mesh,
        compiler_params=pltpu.CompilerParams(needs_layout_passes=False),
        out_type=jax.ShapeDtypeStruct((R, 16), jnp.float32),
        scratch_types=[
            pltpu.VMEM((2, _CHUNK_H, Wd), jnp.float32),
            pltpu.VMEM((_NBINS,), jnp.int32),
            pltpu.VMEM((_NBINS,), jnp.float32),
            pltpu.VMEM((16,), jnp.float32),
            pltpu.SemaphoreType.DMA,
            pltpu.SemaphoreType.DMA,
        ],
    )(functools.partial(_sc_stats_kernel, k=k, rows_per_worker=R // 32,
                        h=H, w=Wd))


def _mlp_kernel(avg_ref, max_ref, w1_ref, b1_ref, w2_ref, b2_ref, scale_ref):
    w1 = w1_ref[...]
    b1 = b1_ref[...]
    w2 = w2_ref[...]
    b2 = b2_ref[...]

    def mlp(p):
        h = jnp.dot(p, w1.T, preferred_element_type=jnp.float32) + b1
        h = jnp.maximum(h, 0.0)
        return jnp.dot(h, w2.T, preferred_element_type=jnp.float32) + b2

    att = mlp(avg_ref[...]) + mlp(max_ref[...])
    scale_ref[...] = jax.nn.sigmoid(att)


def _scale_kernel(x_ref, s_ref, o_ref):
    o_ref[...] = x_ref[...] * s_ref[...]


def kernel(x, W1, b1, W2, b2):
    B, C, H, Wd = x.shape
    N = H * Wd
    R = B * C
    k = int(round(N * _PERCENT_T))
    x3 = x.reshape(R, H, Wd)

    pools = _make_sc_stats(R, H, Wd, k)(x3)

    avg_pool = pools[:, 0].reshape(B, C)
    max_pool = pools[:, 1].reshape(B, C)

    scale = pl.pallas_call(
        _mlp_kernel,
        out_shape=jax.ShapeDtypeStruct((B, C), jnp.float32),
    )(avg_pool, max_pool, W1, b1.reshape(1, -1), W2, b2.reshape(1, -1))

    scale3 = scale.reshape(R, 1, 1)
    hb = _H_BLK if H % _H_BLK == 0 else H
    rows = _ROWS_PER_BLOCK
    y = pl.pallas_call(
        _scale_kernel,
        grid=(R // rows, H // hb),
        in_specs=[
            pl.BlockSpec((rows, hb, Wd), lambda i, j: (i, j, 0)),
            pl.BlockSpec((rows, 1, 1), lambda i, j: (i, 0, 0)),
        ],
        out_specs=pl.BlockSpec((rows, hb, Wd), lambda i, j: (i, j, 0)),
        out_shape=jax.ShapeDtypeStruct((R, H, Wd), jnp.float32),
    )(x3, scale3)

    return y.reshape(B, C, H, Wd)


# speedup vs baseline: 2.9882x; 2.5608x over previous
import functools

import jax
import jax.numpy as jnp
from jax import lax
from jax.experimental import pallas as pl
from jax.experimental.pallas import tpu as pltpu
from jax.experimental.pallas import tpu_sc as plsc

_PERCENT_T = 0.02
_ROWS_PER_BLOCK = 8
_H_BLK = 96

_NBINS = 8192
_BIN_SHIFT = 19
_CHUNK_H = 96


def _sc_stats_kernel(x_hbm, out_hbm, buf, counts, sums, outbuf, sem0, sem1,
                     *, k, rows_per_worker, h, w):
    cid = lax.axis_index("c")
    sid = lax.axis_index("s")
    wid = sid * 2 + cid
    n_chunks = h // _CHUNK_H
    ki = jnp.int32(k)
    ones_i = jnp.ones((16,), jnp.int32)

    def row_body(rr, _):
        row = wid * rows_per_worker + rr

        def dma(c, bref, sem):
            return pltpu.make_async_copy(
                x_hbm.at[row, pl.ds(c * _CHUNK_H, _CHUNK_H), :], bref, sem)

        def zero_body(i):
            counts[pl.ds(i * 16, 16)] = jnp.zeros((16,), jnp.int32)
            sums[pl.ds(i * 16, 16)] = jnp.zeros((16,), jnp.float32)

        plsc.parallel_loop(0, _NBINS // 16, unroll=4)(zero_body)

        def proc(bref, carry):
            def outer_i(i, car):
                rm, bm = car
                for uix in range(w // 16):
                    v = bref[i, pl.ds(uix * 16, 16)]
                    u = plsc.bitcast(v, jnp.int32)
                    s = u ^ ((u >> 31) | jnp.int32(-2147483648))
                    bin_ = lax.shift_right_logical(s, _BIN_SHIFT)
                    plsc.addupdate_scatter(counts, [bin_], ones_i)
                    plsc.addupdate_scatter(sums, [bin_], v)
                    rm = jnp.maximum(rm, v)
                    bm = jnp.maximum(bm, bin_)
                return (rm, bm)

            return lax.fori_loop(0, _CHUNK_H, outer_i, carry)

        dma(0, buf.at[0], sem0).start()

        def pair_body(p, carry):
            base = 2 * p
            dma(base + 1, buf.at[1], sem1).start()
            dma(base, buf.at[0], sem0).wait()
            carry = proc(buf.at[0], carry)

            @pl.when(base + 2 < n_chunks)
            def _():
                dma(base + 2, buf.at[0], sem0).start()

            dma(base + 1, buf.at[1], sem1).wait()
            return proc(buf.at[1], carry)

        init = (jnp.full((16,), -3.0e38, jnp.float32),
                jnp.zeros((16,), jnp.int32))
        runmax, binmax = lax.fori_loop(0, n_chunks // 2, pair_body, init)
        rowmax = lax.reduce_max(runmax, (0,))
        g_start = lax.reduce_max(binmax, (0,)) // 16

        def scan_cond(st):
            g, cnt, sm, pc, ps = st
            return cnt < ki

        def scan_body(st):
            g, cnt, sm, pc, ps = st
            cv = counts[pl.ds(g * 16, 16)]
            sv = sums[pl.ds(g * 16, 16)]
            return (g - 1,
                    cnt + lax.reduce_sum(cv, (0,)),
                    sm + lax.reduce_sum(sv, (0,)),
                    cnt, sm)

        st0 = (g_start, jnp.int32(0), jnp.float32(0.0), jnp.int32(0), jnp.float32(0.0))
        g_end, cnt_t, sm_t, prev_c, prev_s = lax.while_loop(scan_cond, scan_body, st0)

        gb = g_end + 1
        cv = counts[pl.ds(gb * 16, 16)]
        sv = sums[pl.ds(gb * 16, 16)]
        rcv = lax.rev(cv, (0,))
        rsv = lax.rev(sv, (0,))
        ccum = lax.cumsum(rcv, axis=0)
        scum = lax.cumsum(rsv, axis=0)
        crossed = (prev_c + ccum) >= ki
        t = jnp.int32(16) - plsc.all_reduce_population_count(crossed)
        onehot = lax.iota(jnp.int32, 16) == t

        def sel_i(vec):
            return lax.reduce_sum(jnp.where(onehot, vec, 0), (0,))

        def sel_f(vec):
            return lax.reduce_sum(jnp.where(onehot, vec, 0.0), (0,))

        cb = sel_i(rcv)
        sb = sel_f(rsv)
        m_above = prev_c + sel_i(ccum) - cb
        s_above = prev_s + sel_f(scum) - sb
        k2 = (ki - m_above).astype(jnp.float32)

        bmv = jnp.full((16,), sb) / jnp.full((16,), cb.astype(jnp.float32))
        avg_v = (jnp.full((16,), s_above) + jnp.full((16,), k2) * bmv) * (
            jnp.float32(1.0 / k))
        lanes = lax.iota(jnp.int32, 16)
        dbg = [avg_v,
               jnp.full((16,), rowmax),
               jnp.full((16,), g_start.astype(jnp.float32)),
               jnp.full((16,), (g_end + 1).astype(jnp.float32)),
               jnp.full((16,), prev_c.astype(jnp.float32)),
               jnp.full((16,), prev_s),
               jnp.full((16,), cnt_t.astype(jnp.float32)),
               jnp.full((16,), sm_t),
               jnp.full((16,), cb.astype(jnp.float32)),
               jnp.full((16,), sb),
               jnp.full((16,), m_above.astype(jnp.float32)),
               jnp.full((16,), s_above),
               jnp.full((16,), k2)]
        out_v = jnp.zeros((16,), jnp.float32)
        for ln, val in enumerate(dbg):
            out_v = jnp.where(lanes == ln, val, out_v)
        outbuf[...] = out_v
        pltpu.sync_copy(outbuf, out_hbm.at[row])
        return 0

    lax.fori_loop(0, rows_per_worker, row_body, 0)


def _make_sc_stats(R_sc, H, Wd, k):
    mesh = plsc.VectorSubcoreMesh(core_axis_name="c", subcore_axis_name="s")
    return functools.partial(
        pl.kernel,
        mesh=mesh,
        compiler_params=pltpu.CompilerParams(needs_layout_passes=False),
        out_type=jax.ShapeDtypeStruct((R_sc, 16), jnp.float32),
        scratch_types=[
            pltpu.VMEM((2, _CHUNK_H, Wd), jnp.float32),
            pltpu.VMEM((_NBINS,), jnp.int32),
            pltpu.VMEM((_NBINS,), jnp.float32),
            pltpu.VMEM((16,), jnp.float32),
            pltpu.SemaphoreType.DMA,
            pltpu.SemaphoreType.DMA,
        ],
    )(functools.partial(_sc_stats_kernel, k=k, rows_per_worker=R_sc // 32,
                        h=H, w=Wd))


def _tc_stats_kernel(x_ref, out_ref, *, k):
    x = x_ref[...]
    rowmax = jnp.max(x, axis=(1, 2), keepdims=True)
    rowmin = jnp.min(x, axis=(1, 2), keepdims=True)
    lo = rowmin - 1.0
    hi = rowmax
    kf = jnp.float32(k)

    def body(_, carry):
        lo, hi = carry
        mid = 0.5 * (lo + hi)
        m = jnp.sum((x > mid).astype(jnp.float32), axis=(1, 2), keepdims=True)
        pred = m >= kf
        lo = jnp.where(pred, mid, lo)
        hi = jnp.where(pred, hi, mid)
        return lo, hi

    lo, hi = lax.fori_loop(0, 12, body, (lo, hi))
    mask = x > hi
    m_hi = jnp.sum(mask.astype(jnp.float32), axis=(1, 2), keepdims=True)
    s_hi = jnp.sum(jnp.where(mask, x, 0.0), axis=(1, 2), keepdims=True)
    sum_est = s_hi + (kf - m_hi) * 0.5 * (lo + hi)
    avg = sum_est / kf
    out_ref[...] = jnp.concatenate([avg, rowmax], axis=2)[:, 0, :]


def _mlp_kernel(avg_ref, max_ref, w1_ref, b1_ref, w2_ref, b2_ref, scale_ref):
    w1 = w1_ref[...]
    b1 = b1_ref[...]
    w2 = w2_ref[...]
    b2 = b2_ref[...]

    def mlp(p):
        h = jnp.dot(p, w1.T, preferred_element_type=jnp.float32) + b1
        h = jnp.maximum(h, 0.0)
        return jnp.dot(h, w2.T, preferred_element_type=jnp.float32) + b2

    att = mlp(avg_ref[...]) + mlp(max_ref[...])
    scale_ref[...] = jax.nn.sigmoid(att)


def _scale_kernel(x_ref, s_ref, o_ref):
    o_ref[...] = x_ref[...] * s_ref[...]


def kernel(x, W1, b1, W2, b2):
    B, C, H, Wd = x.shape
    N = H * Wd
    R = B * C
    k = int(round(N * _PERCENT_T))
    x3 = x.reshape(R, H, Wd)

    r_sc = 192
    rows = _ROWS_PER_BLOCK
    pools_sc = _make_sc_stats(r_sc, H, Wd, k)(x3)

    n_tc = R - r_sc
    off = r_sc // rows
    pools_tc = pl.pallas_call(
        functools.partial(_tc_stats_kernel, k=k),
        grid=(n_tc // rows,),
        in_specs=[pl.BlockSpec((rows, H, Wd), lambda i: (i + off, 0, 0))],
        out_specs=pl.BlockSpec((rows, 2), lambda i: (i, 0)),
        out_shape=jax.ShapeDtypeStruct((n_tc, 2), jnp.float32),
    )(x3)

    avg_pool = jnp.concatenate([pools_sc[:, 0], pools_tc[:, 0]]).reshape(B, C)
    max_pool = jnp.concatenate([pools_sc[:, 1], pools_tc[:, 1]]).reshape(B, C)

    scale = pl.pallas_call(
        _mlp_kernel,
        out_shape=jax.ShapeDtypeStruct((B, C), jnp.float32),
    )(avg_pool, max_pool, W1, b1.reshape(1, -1), W2, b2.reshape(1, -1))

    scale3 = scale.reshape(R, 1, 1)
    hb = _H_BLK if H % _H_BLK == 0 else H
    rows = _ROWS_PER_BLOCK
    y = pl.pallas_call(
        _scale_kernel,
        grid=(R // rows, H // hb),
        in_specs=[
            pl.BlockSpec((rows, hb, Wd), lambda i, j: (i, j, 0)),
            pl.BlockSpec((rows, 1, 1), lambda i, j: (i, 0, 0)),
        ],
        out_specs=pl.BlockSpec((rows, hb, Wd), lambda i, j: (i, j, 0)),
        out_shape=jax.ShapeDtypeStruct((R, H, Wd), jnp.float32),
    )(x3, scale3)

    return y.reshape(B, C, H, Wd)


# speedup vs baseline: 3.1599x; 1.0575x over previous
import functools

import jax
import jax.numpy as jnp
from jax import lax
from jax.experimental import pallas as pl
from jax.experimental.pallas import tpu as pltpu
from jax.experimental.pallas import tpu_sc as plsc

_PERCENT_T = 0.02
_ROWS_PER_BLOCK = 8
_H_BLK = 96

_NBINS = 8192
_BIN_SHIFT = 19
_CHUNK_H = 96


def _sc_stats_kernel(x_hbm, out_hbm, buf, counts, sums, outbuf, sem0, sem1,
                     *, k, rows_per_worker, h, w):
    cid = lax.axis_index("c")
    sid = lax.axis_index("s")
    wid = sid * 2 + cid
    n_chunks = h // _CHUNK_H
    ki = jnp.int32(k)
    ones_i = jnp.ones((16,), jnp.int32)

    def row_body(rr, _):
        row = wid * rows_per_worker + rr

        def dma(c, bref, sem):
            return pltpu.make_async_copy(
                x_hbm.at[row, pl.ds(c * _CHUNK_H, _CHUNK_H), :], bref, sem)

        def zero_body(i):
            counts[pl.ds(i * 16, 16)] = jnp.zeros((16,), jnp.int32)
            sums[pl.ds(i * 16, 16)] = jnp.zeros((16,), jnp.float32)

        plsc.parallel_loop(0, _NBINS // 16, unroll=4)(zero_body)

        def proc(bref, carry):
            def outer_i(i, car):
                rm, bm = car
                for uix in range(w // 16):
                    v = bref[i, pl.ds(uix * 16, 16)]
                    u = plsc.bitcast(v, jnp.int32)
                    s = u ^ ((u >> 31) | jnp.int32(-2147483648))
                    bin_ = lax.shift_right_logical(s, _BIN_SHIFT)
                    plsc.addupdate_scatter(counts, [bin_], ones_i)
                    plsc.addupdate_scatter(sums, [bin_], v)
                    rm = jnp.maximum(rm, v)
                    bm = jnp.maximum(bm, bin_)
                return (rm, bm)

            return lax.fori_loop(0, _CHUNK_H, outer_i, carry)

        dma(0, buf.at[0], sem0).start()

        def pair_body(p, carry):
            base = 2 * p
            dma(base + 1, buf.at[1], sem1).start()
            dma(base, buf.at[0], sem0).wait()
            carry = proc(buf.at[0], carry)

            @pl.when(base + 2 < n_chunks)
            def _():
                dma(base + 2, buf.at[0], sem0).start()

            dma(base + 1, buf.at[1], sem1).wait()
            return proc(buf.at[1], carry)

        init = (jnp.full((16,), -3.0e38, jnp.float32),
                jnp.zeros((16,), jnp.int32))
        runmax, binmax = lax.fori_loop(0, n_chunks // 2, pair_body, init)
        rowmax = lax.reduce_max(runmax, (0,))
        g_start = lax.reduce_max(binmax, (0,)) // 16

        def scan_cond(st):
            g, cnt, sm, pc, ps = st
            return cnt < ki

        def scan_body(st):
            g, cnt, sm, pc, ps = st
            cv = counts[pl.ds(g * 16, 16)]
            sv = sums[pl.ds(g * 16, 16)]
            return (g - 1,
                    cnt + lax.reduce_sum(cv, (0,)),
                    sm + lax.reduce_sum(sv, (0,)),
                    cnt, sm)

        st0 = (g_start, jnp.int32(0), jnp.float32(0.0), jnp.int32(0), jnp.float32(0.0))
        g_end, cnt_t, sm_t, prev_c, prev_s = lax.while_loop(scan_cond, scan_body, st0)

        gb = g_end + 1
        cv = counts[pl.ds(gb * 16, 16)]
        sv = sums[pl.ds(gb * 16, 16)]
        rcv = lax.rev(cv, (0,))
        rsv = lax.rev(sv, (0,))
        ccum = lax.cumsum(rcv, axis=0)
        scum = lax.cumsum(rsv, axis=0)
        crossed = (prev_c + ccum) >= ki
        t = jnp.int32(16) - plsc.all_reduce_population_count(crossed)
        onehot = lax.iota(jnp.int32, 16) == t

        def sel_i(vec):
            return lax.reduce_sum(jnp.where(onehot, vec, 0), (0,))

        def sel_f(vec):
            return lax.reduce_sum(jnp.where(onehot, vec, 0.0), (0,))

        cb = sel_i(rcv)
        sb = sel_f(rsv)
        m_above = prev_c + sel_i(ccum) - cb
        s_above = prev_s + sel_f(scum) - sb
        k2 = (ki - m_above).astype(jnp.float32)

        bmv = jnp.full((16,), sb) / jnp.full((16,), cb.astype(jnp.float32))
        avg_v = (jnp.full((16,), s_above) + jnp.full((16,), k2) * bmv) * (
            jnp.float32(1.0 / k))
        lanes = lax.iota(jnp.int32, 16)
        dbg = [avg_v,
               jnp.full((16,), rowmax),
               jnp.full((16,), g_start.astype(jnp.float32)),
               jnp.full((16,), (g_end + 1).astype(jnp.float32)),
               jnp.full((16,), prev_c.astype(jnp.float32)),
               jnp.full((16,), prev_s),
               jnp.full((16,), cnt_t.astype(jnp.float32)),
               jnp.full((16,), sm_t),
               jnp.full((16,), cb.astype(jnp.float32)),
               jnp.full((16,), sb),
               jnp.full((16,), m_above.astype(jnp.float32)),
               jnp.full((16,), s_above),
               jnp.full((16,), k2)]
        out_v = jnp.zeros((16,), jnp.float32)
        for ln, val in enumerate(dbg):
            out_v = jnp.where(lanes == ln, val, out_v)
        outbuf[...] = out_v
        pltpu.sync_copy(outbuf, out_hbm.at[row])
        return 0

    lax.fori_loop(0, rows_per_worker, row_body, 0)


def _make_sc_stats(R_sc, H, Wd, k):
    mesh = plsc.VectorSubcoreMesh(core_axis_name="c", subcore_axis_name="s")
    return functools.partial(
        pl.kernel,
        mesh=mesh,
        compiler_params=pltpu.CompilerParams(needs_layout_passes=False),
        out_type=jax.ShapeDtypeStruct((R_sc, 16), jnp.float32),
        scratch_types=[
            pltpu.VMEM((2, _CHUNK_H, Wd), jnp.float32),
            pltpu.VMEM((_NBINS,), jnp.int32),
            pltpu.VMEM((_NBINS,), jnp.float32),
            pltpu.VMEM((16,), jnp.float32),
            pltpu.SemaphoreType.DMA,
            pltpu.SemaphoreType.DMA,
        ],
    )(functools.partial(_sc_stats_kernel, k=k, rows_per_worker=R_sc // 32,
                        h=H, w=Wd))


def _tc_stats_kernel(x_ref, out_ref, *, k):
    x = x_ref[...]
    rowmax = jnp.max(x, axis=(1, 2), keepdims=True)
    rowmin = jnp.min(x, axis=(1, 2), keepdims=True)
    lo = rowmin - 1.0
    hi = rowmax
    kf = jnp.float32(k)

    def body(_, carry):
        lo, hi = carry
        mid = 0.5 * (lo + hi)
        m = jnp.sum((x > mid).astype(jnp.float32), axis=(1, 2), keepdims=True)
        pred = m >= kf
        lo = jnp.where(pred, mid, lo)
        hi = jnp.where(pred, hi, mid)
        return lo, hi

    lo, hi = lax.fori_loop(0, 10, body, (lo, hi))
    mask = x > hi
    m_hi = jnp.sum(mask.astype(jnp.float32), axis=(1, 2), keepdims=True)
    s_hi = jnp.sum(jnp.where(mask, x, 0.0), axis=(1, 2), keepdims=True)
    sum_est = s_hi + (kf - m_hi) * 0.5 * (lo + hi)
    avg = sum_est / kf
    out_ref[...] = jnp.concatenate([avg, rowmax], axis=2)[:, 0, :]


def _mlp_kernel(avg_ref, max_ref, w1_ref, b1_ref, w2_ref, b2_ref, scale_ref):
    w1 = w1_ref[...]
    b1 = b1_ref[...]
    w2 = w2_ref[...]
    b2 = b2_ref[...]

    def mlp(p):
        h = jnp.dot(p, w1.T, preferred_element_type=jnp.float32) + b1
        h = jnp.maximum(h, 0.0)
        return jnp.dot(h, w2.T, preferred_element_type=jnp.float32) + b2

    att = mlp(avg_ref[...]) + mlp(max_ref[...])
    scale_ref[...] = jax.nn.sigmoid(att)


def _scale_kernel(x_ref, s_ref, o_ref):
    o_ref[...] = x_ref[...] * s_ref[...]


def kernel(x, W1, b1, W2, b2):
    B, C, H, Wd = x.shape
    N = H * Wd
    R = B * C
    k = int(round(N * _PERCENT_T))
    x3 = x.reshape(R, H, Wd)

    r_sc = 160
    rows = _ROWS_PER_BLOCK
    pools_sc = _make_sc_stats(r_sc, H, Wd, k)(x3)

    n_tc = R - r_sc
    off = r_sc // rows
    pools_tc = pl.pallas_call(
        functools.partial(_tc_stats_kernel, k=k),
        grid=(n_tc // rows,),
        in_specs=[pl.BlockSpec((rows, H, Wd), lambda i: (i + off, 0, 0))],
        out_specs=pl.BlockSpec((rows, 2), lambda i: (i, 0)),
        out_shape=jax.ShapeDtypeStruct((n_tc, 2), jnp.float32),
    )(x3)

    avg_pool = jnp.concatenate([pools_sc[:, 0], pools_tc[:, 0]]).reshape(B, C)
    max_pool = jnp.concatenate([pools_sc[:, 1], pools_tc[:, 1]]).reshape(B, C)

    scale = pl.pallas_call(
        _mlp_kernel,
        out_shape=jax.ShapeDtypeStruct((B, C), jnp.float32),
    )(avg_pool, max_pool, W1, b1.reshape(1, -1), W2, b2.reshape(1, -1))

    scale3 = scale.reshape(R, 1, 1)
    hb = _H_BLK if H % _H_BLK == 0 else H
    rows = _ROWS_PER_BLOCK
    y = pl.pallas_call(
        _scale_kernel,
        grid=(R // rows, H // hb),
        in_specs=[
            pl.BlockSpec((rows, hb, Wd), lambda i, j: (i, j, 0)),
            pl.BlockSpec((rows, 1, 1), lambda i, j: (i, 0, 0)),
        ],
        out_specs=pl.BlockSpec((rows, hb, Wd), lambda i, j: (i, j, 0)),
        out_shape=jax.ShapeDtypeStruct((R, H, Wd), jnp.float32),
    )(x3, scale3)

    return y.reshape(B, C, H, Wd)
